# Initial kernel scaffold; baseline (speedup 1.0000x reference)
import jax
import jax.numpy as jnp
from jax.experimental import pallas as pl


def kernel(interaction_mu, interaction_log_sigma, conc_mu, conc_log_sigma, eps_energy, eps_conc, family_ids):
    raise NotImplementedError("write your pallas kernel here")


# speedup vs baseline: 1.1101x; 1.1101x over previous
import functools

import jax
import jax.numpy as jnp
from jax import lax
from jax.experimental import pallas as pl
from jax.experimental.pallas import tpu as pltpu
from jax.experimental.pallas import tpu_sc as plsc

B = 4096
U = 256
F = 1000
D = 2 * U
ROW = 2 * D
FPAD = 1024

NC, NS = 2, 16
NW = NC * NS
BPW = B // NW
CH = 32
NCH = BPW // CH
VL = 16


def _prep_body(mu0_ref, mu1_ref, ls0_ref, ls1_ref, table_ref):
    u_iota = lax.broadcasted_iota(jnp.int32, (U, D), 0)
    j_iota = lax.broadcasted_iota(jnp.int32, (U, D), 1)
    pa = (j_iota == 2 * u_iota).astype(jnp.float32)
    pb = (j_iota == 2 * u_iota + 1).astype(jnp.float32)
    dn = (((0,), (0,)), ((), ()))

    def t_interleave(a, b):
        return (lax.dot_general(a, pa, dn, preferred_element_type=jnp.float32)
                + lax.dot_general(b, pb, dn, preferred_element_type=jnp.float32))

    table_ref[:, :D] = t_interleave(mu0_ref[...], mu1_ref[...])
    table_ref[:, D:] = jnp.exp(t_interleave(ls0_ref[...], ls1_ref[...]))


def _prep(mu0, mu1, ls0, ls1):
    return pl.pallas_call(
        _prep_body,
        out_shape=jax.ShapeDtypeStruct((F, ROW), jnp.float32),
    )(mu0, mu1, ls0, ls1)


_sc_mesh = plsc.VectorSubcoreMesh(core_axis_name="c", subcore_axis_name="s")


@functools.partial(
    pl.kernel,
    out_type=(
        jax.ShapeDtypeStruct((B, D), jnp.float32),
        jax.ShapeDtypeStruct((B,), jnp.float32),
    ),
    mesh=_sc_mesh,
    compiler_params=pltpu.CompilerParams(needs_layout_passes=False),
    scratch_types=[
        pltpu.VMEM((NCH, CH), jnp.int32),
        pltpu.VMEM((CH, ROW), jnp.float32),
        pltpu.VMEM((CH, D), jnp.float32),
        pltpu.VMEM((CH, D), jnp.float32),
        pltpu.VMEM((FPAD,), jnp.float32),
        pltpu.VMEM((FPAD,), jnp.float32),
        pltpu.VMEM((BPW,), jnp.float32),
        pltpu.VMEM((BPW,), jnp.float32),
        pltpu.SemaphoreType.DMA,
    ],
)
def _sc_sample(table_hbm, ids_hbm, eps_hbm, cmu_hbm, cls_hbm, epsc_hbm,
               energies_hbm, conc_hbm,
               ids_v, rows_v, eps_v, out_v, cmu_v, cls_v, epsc_v, conc_v, sem):
    wid = lax.axis_index("s") * NC + lax.axis_index("c")
    base = wid * BPW

    pltpu.sync_copy(ids_hbm.at[wid], ids_v)
    pltpu.sync_copy(cmu_hbm, cmu_v)
    pltpu.sync_copy(cls_hbm, cls_v)
    pltpu.sync_copy(epsc_hbm.at[pl.ds(base, BPW)], epsc_v)

    for t in range(BPW // VL):
        ids16 = ids_v[(t * VL) // CH, pl.ds((t * VL) % CH, VL)]
        cm = plsc.load_gather(cmu_v, [ids16])
        cs = jnp.exp(plsc.load_gather(cls_v, [ids16]))
        ec = epsc_v[pl.ds(t * VL, VL)]
        conc_v[pl.ds(t * VL, VL)] = jnp.exp(cm + cs * ec)
    pltpu.sync_copy(conc_v, conc_hbm.at[pl.ds(base, BPW)])

    for c in range(NCH):
        pltpu.async_copy(table_hbm.at[ids_v.at[c]], rows_v, sem).wait()
        pltpu.sync_copy(eps_hbm.at[pl.ds(base + c * CH, CH)], eps_v)

        def fma_body(k, _):
            i = k // (D // VL)
            j = (k % (D // VL)) * VL
            mu = rows_v[i, pl.ds(j, VL)]
            sg = rows_v[i, pl.ds(D + j, VL)]
            ep = eps_v[i, pl.ds(j, VL)]
            out_v[i, pl.ds(j, VL)] = mu + sg * ep
            return 0

        lax.fori_loop(0, CH * (D // VL), fma_body, 0)
        pltpu.sync_copy(out_v, energies_hbm.at[pl.ds(base + c * CH, CH)])


def kernel(interaction_mu, interaction_log_sigma, conc_mu, conc_log_sigma,
           eps_energy, eps_conc, family_ids):
    mu0 = interaction_mu[:, :, 0]
    mu1 = interaction_mu[:, :, 1]
    ls0 = interaction_log_sigma[:, :, 0]
    ls1 = interaction_log_sigma[:, :, 1]
    table = _prep(mu0, mu1, ls0, ls1)

    ids3 = family_ids.reshape(NW, NCH, CH)
    eps2 = eps_energy.reshape(B, D)
    cmu_p = jnp.zeros((FPAD,), jnp.float32).at[:F].set(conc_mu)
    cls_p = jnp.zeros((FPAD,), jnp.float32).at[:F].set(conc_log_sigma)

    energies2, conc = _sc_sample(table, ids3, eps2, cmu_p, cls_p, eps_conc)
    return energies2.reshape(B, U, 2), conc, family_ids
